# baseline (speedup 1.0000x reference)
import functools

import jax
import jax.numpy as jnp
from jax import lax
from jax.experimental import pallas as pl
from jax.experimental.pallas import tpu as pltpu

_EPS = 1e-5
_CHUNK = 2048
_NCORES = 2


def _round_up(n, m):
    return -(-n // m) * m


def _ln_folded(v, gamma, beta):
    d = v.shape[-1]
    s1 = jnp.sum(v, axis=-1, keepdims=True)
    s2 = jnp.sum(v * v, axis=-1, keepdims=True)
    mu = s1 * (1.0 / d)
    inv = lax.rsqrt((s2 * (1.0 / d) - mu * mu) + _EPS)
    return (inv * v) * gamma - (inv * mu) * gamma + beta


def _pipeline_kernel(x_hbm, c_hbm, gx_ref, bx_ref, gc_ref, bc_ref,
                     wx_ref, wc_ref, bo_ref, o_hbm,
                     xbuf, cbuf, obuf, xsem, csem, osem,
                     *, chunk, nchunks):
    core = pl.program_id(0)
    base = core * (nchunks * chunk)

    def start_in(slot, step):
        row = base + step * chunk
        pltpu.make_async_copy(x_hbm.at[pl.ds(row, chunk), :],
                              xbuf.at[slot], xsem.at[slot]).start()
        pltpu.make_async_copy(c_hbm.at[pl.ds(row, chunk), :],
                              cbuf.at[slot], csem.at[slot]).start()

    def wait_in(slot):
        pltpu.make_async_copy(xbuf.at[slot], xbuf.at[slot],
                              xsem.at[slot]).wait()
        pltpu.make_async_copy(cbuf.at[slot], cbuf.at[slot],
                              csem.at[slot]).wait()

    def start_out(slot, step):
        row = base + step * chunk
        pltpu.make_async_copy(obuf.at[slot], o_hbm.at[pl.ds(row, chunk), :],
                              osem.at[slot]).start()

    def wait_out(slot):
        pltpu.make_async_copy(obuf.at[slot], obuf.at[slot],
                              osem.at[slot]).wait()

    start_in(0, 0)
    if nchunks > 1:
        start_in(1, 1)

    for step in range(nchunks):
        islot = step % 3
        oslot = step % 2
        if step + 2 < nchunks:
            start_in((step + 2) % 3, step + 2)
        wait_in(islot)
        if step >= 2:
            wait_out(oslot)
        y = _ln_folded(xbuf[islot], gx_ref[...], bx_ref[...])
        z = _ln_folded(cbuf[islot], gc_ref[...], bc_ref[...])
        obuf[oslot] = (
            jnp.dot(y, wx_ref[...], preferred_element_type=jnp.float32)
            + jnp.dot(z, wc_ref[...], preferred_element_type=jnp.float32)
            + bo_ref[...])
        start_out(oslot, step)

    if nchunks == 1:
        wait_out(0)
    else:
        wait_out((nchunks - 2) % 2)
        wait_out((nchunks - 1) % 2)


def kernel(x, context, norm_w, norm_b, ctx_w, ctx_b, Wx, Wc, b_out):
    *lead, dim = x.shape
    cdim = context.shape[-1]
    out_dim = Wx.shape[1]

    x2 = x.reshape(-1, dim)
    c2 = context.reshape(-1, cdim)
    rows = x2.shape[0]

    chunk = min(_CHUNK, _round_up(rows, 8))
    rows_p = _round_up(rows, _NCORES * chunk)
    if rows_p != rows:
        x2 = jnp.pad(x2, ((0, rows_p - rows), (0, 0)))
        c2 = jnp.pad(c2, ((0, rows_p - rows), (0, 0)))
    nchunks = rows_p // (_NCORES * chunk)

    body = functools.partial(_pipeline_kernel, chunk=chunk, nchunks=nchunks)
    out = pl.pallas_call(
        body,
        out_shape=jax.ShapeDtypeStruct((rows_p, out_dim), x.dtype),
        grid_spec=pltpu.PrefetchScalarGridSpec(
            num_scalar_prefetch=0,
            grid=(_NCORES,),
            in_specs=[
                pl.BlockSpec(memory_space=pl.ANY),
                pl.BlockSpec(memory_space=pl.ANY),
                pl.BlockSpec((1, dim), lambda i: (0, 0)),
                pl.BlockSpec((1, dim), lambda i: (0, 0)),
                pl.BlockSpec((1, cdim), lambda i: (0, 0)),
                pl.BlockSpec((1, cdim), lambda i: (0, 0)),
                pl.BlockSpec((dim, out_dim), lambda i: (0, 0)),
                pl.BlockSpec((cdim, out_dim), lambda i: (0, 0)),
                pl.BlockSpec((1, out_dim), lambda i: (0, 0)),
            ],
            out_specs=pl.BlockSpec(memory_space=pl.ANY),
            scratch_shapes=[
                pltpu.VMEM((3, chunk, dim), jnp.float32),
                pltpu.VMEM((3, chunk, cdim), jnp.float32),
                pltpu.VMEM((2, chunk, out_dim), jnp.float32),
                pltpu.SemaphoreType.DMA((3,)),
                pltpu.SemaphoreType.DMA((3,)),
                pltpu.SemaphoreType.DMA((2,)),
            ],
        ),
        compiler_params=pltpu.CompilerParams(
            dimension_semantics=("parallel",),
            vmem_limit_bytes=56 << 20),
    )(x2, c2,
      norm_w.reshape(1, dim).astype(jnp.float32),
      norm_b.reshape(1, dim).astype(jnp.float32),
      ctx_w.reshape(1, cdim).astype(jnp.float32),
      ctx_b.reshape(1, cdim).astype(jnp.float32),
      Wx.astype(jnp.float32), Wc.astype(jnp.float32),
      b_out.reshape(1, out_dim).astype(jnp.float32))
    return out[:rows].reshape(*lead, out_dim)


# speedup vs baseline: 1.1010x; 1.1010x over previous
import functools

import jax
import jax.numpy as jnp
from jax import lax
from jax.experimental import pallas as pl
from jax.experimental.pallas import tpu as pltpu

_EPS = 1e-5
_CHUNK = 1024
_NCORES = 2
_IBUF = 4
_OBUF = 3


def _round_up(n, m):
    return -(-n // m) * m


def _ln_folded(v, gamma, beta):
    d = v.shape[-1]
    s1 = jnp.sum(v, axis=-1, keepdims=True)
    s2 = jnp.sum(v * v, axis=-1, keepdims=True)
    mu = s1 * (1.0 / d)
    inv = lax.rsqrt((s2 * (1.0 / d) - mu * mu) + _EPS)
    return (inv * v) * gamma - (inv * mu) * gamma + beta


def _pipeline_kernel(x_hbm, c_hbm, gx_ref, bx_ref, gc_ref, bc_ref,
                     wx_ref, wc_ref, bo_ref, o_hbm,
                     xbuf, cbuf, obuf, xsem, csem, osem,
                     *, chunk, nchunks):
    core = pl.program_id(0)
    base = core * (nchunks * chunk)

    def start_in(slot, step):
        row = base + step * chunk
        pltpu.make_async_copy(x_hbm.at[pl.ds(row, chunk), :],
                              xbuf.at[slot], xsem.at[slot]).start()
        pltpu.make_async_copy(c_hbm.at[pl.ds(row, chunk), :],
                              cbuf.at[slot], csem.at[slot]).start()

    def wait_in(slot):
        pltpu.make_async_copy(xbuf.at[slot], xbuf.at[slot],
                              xsem.at[slot]).wait()
        pltpu.make_async_copy(cbuf.at[slot], cbuf.at[slot],
                              csem.at[slot]).wait()

    def start_out(slot, step):
        row = base + step * chunk
        pltpu.make_async_copy(obuf.at[slot], o_hbm.at[pl.ds(row, chunk), :],
                              osem.at[slot]).start()

    def wait_out(slot):
        pltpu.make_async_copy(obuf.at[slot], obuf.at[slot],
                              osem.at[slot]).wait()

    pf = _IBUF - 1
    for k in range(min(pf, nchunks)):
        start_in(k % _IBUF, k)

    for step in range(nchunks):
        islot = step % _IBUF
        oslot = step % _OBUF
        if step + pf < nchunks:
            start_in((step + pf) % _IBUF, step + pf)
        wait_in(islot)
        if step >= _OBUF:
            wait_out(oslot)
        y = _ln_folded(xbuf[islot], gx_ref[...], bx_ref[...])
        z = _ln_folded(cbuf[islot], gc_ref[...], bc_ref[...])
        obuf[oslot] = (
            jnp.dot(y, wx_ref[...], preferred_element_type=jnp.float32)
            + jnp.dot(z, wc_ref[...], preferred_element_type=jnp.float32)
            + bo_ref[...])
        start_out(oslot, step)

    for s in range(max(0, nchunks - _OBUF), nchunks):
        wait_out(s % _OBUF)


def kernel(x, context, norm_w, norm_b, ctx_w, ctx_b, Wx, Wc, b_out):
    *lead, dim = x.shape
    cdim = context.shape[-1]
    out_dim = Wx.shape[1]

    x2 = x.reshape(-1, dim)
    c2 = context.reshape(-1, cdim)
    rows = x2.shape[0]

    chunk = min(_CHUNK, _round_up(rows, 8))
    rows_p = _round_up(rows, _NCORES * chunk)
    if rows_p != rows:
        x2 = jnp.pad(x2, ((0, rows_p - rows), (0, 0)))
        c2 = jnp.pad(c2, ((0, rows_p - rows), (0, 0)))
    nchunks = rows_p // (_NCORES * chunk)

    body = functools.partial(_pipeline_kernel, chunk=chunk, nchunks=nchunks)
    out = pl.pallas_call(
        body,
        out_shape=jax.ShapeDtypeStruct((rows_p, out_dim), x.dtype),
        grid_spec=pltpu.PrefetchScalarGridSpec(
            num_scalar_prefetch=0,
            grid=(_NCORES,),
            in_specs=[
                pl.BlockSpec(memory_space=pl.ANY),
                pl.BlockSpec(memory_space=pl.ANY),
                pl.BlockSpec((1, dim), lambda i: (0, 0)),
                pl.BlockSpec((1, dim), lambda i: (0, 0)),
                pl.BlockSpec((1, cdim), lambda i: (0, 0)),
                pl.BlockSpec((1, cdim), lambda i: (0, 0)),
                pl.BlockSpec((dim, out_dim), lambda i: (0, 0)),
                pl.BlockSpec((cdim, out_dim), lambda i: (0, 0)),
                pl.BlockSpec((1, out_dim), lambda i: (0, 0)),
            ],
            out_specs=pl.BlockSpec(memory_space=pl.ANY),
            scratch_shapes=[
                pltpu.VMEM((_IBUF, chunk, dim), jnp.float32),
                pltpu.VMEM((_IBUF, chunk, cdim), jnp.float32),
                pltpu.VMEM((_OBUF, chunk, out_dim), jnp.float32),
                pltpu.SemaphoreType.DMA((_IBUF,)),
                pltpu.SemaphoreType.DMA((_IBUF,)),
                pltpu.SemaphoreType.DMA((_OBUF,)),
            ],
        ),
        compiler_params=pltpu.CompilerParams(
            dimension_semantics=("parallel",),
            vmem_limit_bytes=56 << 20),
    )(x2, c2,
      norm_w.reshape(1, dim).astype(jnp.float32),
      norm_b.reshape(1, dim).astype(jnp.float32),
      ctx_w.reshape(1, cdim).astype(jnp.float32),
      ctx_b.reshape(1, cdim).astype(jnp.float32),
      Wx.astype(jnp.float32), Wc.astype(jnp.float32),
      b_out.reshape(1, out_dim).astype(jnp.float32))
    return out[:rows].reshape(*lead, out_dim)


# speedup vs baseline: 1.1138x; 1.0116x over previous
import functools

import jax
import jax.numpy as jnp
from jax import lax
from jax.experimental import pallas as pl
from jax.experimental.pallas import tpu as pltpu

_EPS = 1e-5
_CHUNK = 1024
_NCORES = 2
_IBUF = 6
_OBUF = 4


def _round_up(n, m):
    return -(-n // m) * m


def _ln_folded(v, gamma, beta):
    d = v.shape[-1]
    s1 = jnp.sum(v, axis=-1, keepdims=True)
    s2 = jnp.sum(v * v, axis=-1, keepdims=True)
    mu = s1 * (1.0 / d)
    inv = lax.rsqrt((s2 * (1.0 / d) - mu * mu) + _EPS)
    return (inv * v) * gamma - (inv * mu) * gamma + beta


def _pipeline_kernel(x_hbm, c_hbm, gx_ref, bx_ref, gc_ref, bc_ref,
                     wx_ref, wc_ref, bo_ref, o_hbm,
                     xbuf, cbuf, obuf, xsem, csem, osem,
                     *, chunk, nchunks):
    core = pl.program_id(0)
    base = core * (nchunks * chunk)

    def start_in(slot, step):
        row = base + step * chunk
        pltpu.make_async_copy(x_hbm.at[pl.ds(row, chunk), :],
                              xbuf.at[slot], xsem.at[slot]).start()
        pltpu.make_async_copy(c_hbm.at[pl.ds(row, chunk), :],
                              cbuf.at[slot], csem.at[slot]).start()

    def wait_in(slot):
        pltpu.make_async_copy(xbuf.at[slot], xbuf.at[slot],
                              xsem.at[slot]).wait()
        pltpu.make_async_copy(cbuf.at[slot], cbuf.at[slot],
                              csem.at[slot]).wait()

    def start_out(slot, step):
        row = base + step * chunk
        pltpu.make_async_copy(obuf.at[slot], o_hbm.at[pl.ds(row, chunk), :],
                              osem.at[slot]).start()

    def wait_out(slot):
        pltpu.make_async_copy(obuf.at[slot], obuf.at[slot],
                              osem.at[slot]).wait()

    pf = _IBUF - 1
    for k in range(min(pf, nchunks)):
        start_in(k % _IBUF, k)

    for step in range(nchunks):
        islot = step % _IBUF
        oslot = step % _OBUF
        if step + pf < nchunks:
            start_in((step + pf) % _IBUF, step + pf)
        wait_in(islot)
        if step >= _OBUF:
            wait_out(oslot)
        y = _ln_folded(xbuf[islot], gx_ref[...], bx_ref[...])
        z = _ln_folded(cbuf[islot], gc_ref[...], bc_ref[...])
        obuf[oslot] = (
            jnp.dot(y, wx_ref[...], preferred_element_type=jnp.float32)
            + jnp.dot(z, wc_ref[...], preferred_element_type=jnp.float32)
            + bo_ref[...])
        start_out(oslot, step)

    for s in range(max(0, nchunks - _OBUF), nchunks):
        wait_out(s % _OBUF)


def kernel(x, context, norm_w, norm_b, ctx_w, ctx_b, Wx, Wc, b_out):
    *lead, dim = x.shape
    cdim = context.shape[-1]
    out_dim = Wx.shape[1]

    x2 = x.reshape(-1, dim)
    c2 = context.reshape(-1, cdim)
    rows = x2.shape[0]

    chunk = min(_CHUNK, _round_up(rows, 8))
    rows_p = _round_up(rows, _NCORES * chunk)
    if rows_p != rows:
        x2 = jnp.pad(x2, ((0, rows_p - rows), (0, 0)))
        c2 = jnp.pad(c2, ((0, rows_p - rows), (0, 0)))
    nchunks = rows_p // (_NCORES * chunk)

    body = functools.partial(_pipeline_kernel, chunk=chunk, nchunks=nchunks)
    out = pl.pallas_call(
        body,
        out_shape=jax.ShapeDtypeStruct((rows_p, out_dim), x.dtype),
        grid_spec=pltpu.PrefetchScalarGridSpec(
            num_scalar_prefetch=0,
            grid=(_NCORES,),
            in_specs=[
                pl.BlockSpec(memory_space=pl.ANY),
                pl.BlockSpec(memory_space=pl.ANY),
                pl.BlockSpec((1, dim), lambda i: (0, 0)),
                pl.BlockSpec((1, dim), lambda i: (0, 0)),
                pl.BlockSpec((1, cdim), lambda i: (0, 0)),
                pl.BlockSpec((1, cdim), lambda i: (0, 0)),
                pl.BlockSpec((dim, out_dim), lambda i: (0, 0)),
                pl.BlockSpec((cdim, out_dim), lambda i: (0, 0)),
                pl.BlockSpec((1, out_dim), lambda i: (0, 0)),
            ],
            out_specs=pl.BlockSpec(memory_space=pl.ANY),
            scratch_shapes=[
                pltpu.VMEM((_IBUF, chunk, dim), jnp.float32),
                pltpu.VMEM((_IBUF, chunk, cdim), jnp.float32),
                pltpu.VMEM((_OBUF, chunk, out_dim), jnp.float32),
                pltpu.SemaphoreType.DMA((_IBUF,)),
                pltpu.SemaphoreType.DMA((_IBUF,)),
                pltpu.SemaphoreType.DMA((_OBUF,)),
            ],
        ),
        compiler_params=pltpu.CompilerParams(
            dimension_semantics=("parallel",),
            vmem_limit_bytes=56 << 20),
    )(x2, c2,
      norm_w.reshape(1, dim).astype(jnp.float32),
      norm_b.reshape(1, dim).astype(jnp.float32),
      ctx_w.reshape(1, cdim).astype(jnp.float32),
      ctx_b.reshape(1, cdim).astype(jnp.float32),
      Wx.astype(jnp.float32), Wc.astype(jnp.float32),
      b_out.reshape(1, out_dim).astype(jnp.float32))
    return out[:rows].reshape(*lead, out_dim)
